# baseline (speedup 1.0000x reference)
import jax
import jax.numpy as jnp
from jax.experimental import pallas as pl

_B, _S, _D, _H = 16, 8192, 128, 4


def _dnc_body(mem_ref, ww_ref, wv_ref, ev_ref, keys_ref, st_ref, out_ref):
    mem = mem_ref[0]
    wwT = ww_ref[0]
    wvT = wv_ref[0].reshape(_D, 1)
    evT = ev_ref[0].reshape(_D, 1)
    keys = keys_ref[0]
    st = st_ref[0]

    kn = jnp.sqrt(jnp.sum(keys * keys, axis=0, keepdims=True))
    kn = jnp.maximum(kn, 1e-30)
    keys2 = keys * (st / kn)

    eye = jnp.eye(_D, dtype=jnp.float32)
    memT = jax.lax.dot_general(
        eye, mem, (((1,), (1,)), ((), ())),
        precision=jax.lax.Precision.HIGHEST,
        preferred_element_type=jnp.float32)

    uT = memT + wwT * (wvT - memT * evT)
    usqT = uT * uT

    dot2T = jax.lax.dot_general(
        keys2, uT, (((0,), (0,)), ((), ())),
        precision=jax.lax.Precision.HIGHEST,
        preferred_element_type=jnp.float32)
    ones = jnp.ones((1, _D), dtype=jnp.float32)
    sumsqT = jnp.dot(ones, usqT, precision=jax.lax.Precision.HIGHEST,
                     preferred_element_type=jnp.float32)

    rs = jax.lax.rsqrt(sumsqT + 1e-30)
    scoresT = dot2T * rs
    e = jnp.exp(scoresT)
    denom = jnp.sum(e, axis=1, keepdims=True)
    wT = e * (1.0 / denom)

    out_ref[0, 0:_D, :] = uT
    out_ref[0, _D:_D + _H, :] = wT


def kernel(memory_matrix, write_weight, write_vector, erase_vector, keys, strengths):
    out_t = pl.pallas_call(
        _dnc_body,
        grid=(_B,),
        in_specs=[
            pl.BlockSpec((1, _S, _D), lambda b: (b, 0, 0)),
            pl.BlockSpec((1, 1, _S), lambda b: (b, 0, 0)),
            pl.BlockSpec((1, 1, _D), lambda b: (b, 0, 0)),
            pl.BlockSpec((1, 1, _D), lambda b: (b, 0, 0)),
            pl.BlockSpec((1, _D, _H), lambda b: (b, 0, 0)),
            pl.BlockSpec((1, 1, _H), lambda b: (b, 0, 0)),
        ],
        out_specs=pl.BlockSpec((1, _D + _H, _S), lambda b: (b, 0, 0)),
        out_shape=jax.ShapeDtypeStruct((_B, _D + _H, _S), jnp.float32),
    )(
        memory_matrix,
        write_weight[:, None, :],
        write_vector[:, None, :],
        erase_vector[:, None, :],
        keys,
        strengths[:, None, :],
    )
    return jnp.transpose(out_t, (0, 2, 1))


# speedup vs baseline: 1.6335x; 1.6335x over previous
import jax
import jax.numpy as jnp
from jax.experimental import pallas as pl

_B, _S, _D, _H = 16, 8192, 128, 4


def _dnc_body(mem_ref, ww_ref, wv_ref, ev_ref, keys_ref, st_ref, out_ref):
    mem = mem_ref[0]
    wwT = ww_ref[0]
    wvT = wv_ref[0].reshape(_D, 1)
    evT = ev_ref[0].reshape(_D, 1)
    keys = keys_ref[0]
    st = st_ref[0]

    kn = jnp.sqrt(jnp.sum(keys * keys, axis=0, keepdims=True))
    kn = jnp.maximum(kn, 1e-30)
    keys2 = keys * (st / kn)

    memT = jnp.transpose(mem)

    uT = memT + wwT * (wvT - memT * evT)
    usqT = uT * uT

    dot2T = jax.lax.dot_general(
        keys2, uT, (((0,), (0,)), ((), ())),
        preferred_element_type=jnp.float32)
    ones = jnp.ones((1, _D), dtype=jnp.float32)
    sumsqT = jnp.dot(ones, usqT, preferred_element_type=jnp.float32)

    rs = jax.lax.rsqrt(sumsqT + 1e-30)
    scoresT = dot2T * rs
    e = jnp.exp(scoresT)
    denom = jnp.sum(e, axis=1, keepdims=True)
    wT = e * (1.0 / denom)

    out_ref[0, 0:_D, :] = uT
    out_ref[0, _D:_D + _H, :] = wT


def kernel(memory_matrix, write_weight, write_vector, erase_vector, keys, strengths):
    out_t = pl.pallas_call(
        _dnc_body,
        grid=(_B,),
        in_specs=[
            pl.BlockSpec((1, _S, _D), lambda b: (b, 0, 0)),
            pl.BlockSpec((1, 1, _S), lambda b: (b, 0, 0)),
            pl.BlockSpec((1, 1, _D), lambda b: (b, 0, 0)),
            pl.BlockSpec((1, 1, _D), lambda b: (b, 0, 0)),
            pl.BlockSpec((1, _D, _H), lambda b: (b, 0, 0)),
            pl.BlockSpec((1, 1, _H), lambda b: (b, 0, 0)),
        ],
        out_specs=pl.BlockSpec((1, _D + _H, _S), lambda b: (b, 0, 0)),
        out_shape=jax.ShapeDtypeStruct((_B, _D + _H, _S), jnp.float32),
    )(
        memory_matrix,
        write_weight[:, None, :],
        write_vector[:, None, :],
        erase_vector[:, None, :],
        keys,
        strengths[:, None, :],
    )
    return jnp.transpose(out_t, (0, 2, 1))


# speedup vs baseline: 1.6342x; 1.0004x over previous
import jax
import jax.numpy as jnp
from jax.experimental import pallas as pl

_B, _S, _D, _H = 16, 8192, 128, 4


def _dnc_body(mem_ref, ww_ref, wv_ref, ev_ref, keys_ref, st_ref, out_ref):
    b = pl.program_id(0)
    mem = mem_ref[0]
    wwT = ww_ref[b]
    wvT = wv_ref[b].reshape(_D, 1)
    evT = ev_ref[b].reshape(_D, 1)
    keys = keys_ref[b]
    st = st_ref[b]

    kn = jnp.sqrt(jnp.sum(keys * keys, axis=0, keepdims=True))
    kn = jnp.maximum(kn, 1e-30)
    keys2 = keys * (st / kn)

    memT = jnp.transpose(mem)

    uT = memT + wwT * (wvT - memT * evT)
    usqT = uT * uT

    dot2T = jax.lax.dot_general(
        keys2, uT, (((0,), (0,)), ((), ())),
        preferred_element_type=jnp.float32)
    ones = jnp.ones((1, _D), dtype=jnp.float32)
    sumsqT = jnp.dot(ones, usqT, preferred_element_type=jnp.float32)

    rs = jax.lax.rsqrt(sumsqT + 1e-30)
    scoresT = dot2T * rs
    e = jnp.exp(scoresT)
    denom = jnp.sum(e, axis=1, keepdims=True)
    wT = e * (1.0 / denom)

    out_ref[0, 0:_D, :] = uT
    out_ref[0, _D:_D + _H, :] = wT


def kernel(memory_matrix, write_weight, write_vector, erase_vector, keys, strengths):
    out_t = pl.pallas_call(
        _dnc_body,
        grid=(_B,),
        in_specs=[
            pl.BlockSpec((1, _S, _D), lambda b: (b, 0, 0)),
            pl.BlockSpec((_B, 1, _S), lambda b: (0, 0, 0)),
            pl.BlockSpec((_B, 1, _D), lambda b: (0, 0, 0)),
            pl.BlockSpec((_B, 1, _D), lambda b: (0, 0, 0)),
            pl.BlockSpec((_B, _D, _H), lambda b: (0, 0, 0)),
            pl.BlockSpec((_B, 1, _H), lambda b: (0, 0, 0)),
        ],
        out_specs=pl.BlockSpec((1, _D + _H, _S), lambda b: (b, 0, 0)),
        out_shape=jax.ShapeDtypeStruct((_B, _D + _H, _S), jnp.float32),
    )(
        memory_matrix,
        write_weight[:, None, :],
        write_vector[:, None, :],
        erase_vector[:, None, :],
        keys,
        strengths[:, None, :],
    )
    return jnp.transpose(out_t, (0, 2, 1))
